# baseline (speedup 1.0000x reference)
import functools

import jax
import jax.numpy as jnp
from jax import lax
from jax.experimental import pallas as pl
from jax.experimental.pallas import tpu as pltpu
from jax.experimental.pallas import tpu_sc as plsc

N = 10000
NPAD = 10240
E = 320000
ETOT = E + N
EPAD = 335872
NC, NS = 2, 16
NW = NC * NS
EPW = EPAD // NW
EPS = EPAD // NS
CH = 128
BLK = 1024
NB = NPAD // BLK
F32 = jnp.float32
I32 = jnp.int32

_mesh = plsc.VectorSubcoreMesh(core_axis_name="c", subcore_axis_name="s")

_NSLICE = NPAD // NS


def _zero_vec(ref, nwords):
    z = jnp.zeros((16,), F32)

    def body(i, _):
        ref[pl.ds(i * 16, 16)] = z
        return 0

    lax.fori_loop(0, nwords // 16, body, 0)


def _zero_rows(ref, nrows, f):
    z = jnp.zeros((16,), F32)

    def body(i, _):
        for k in range(f // 16):
            ref[i, pl.ds(k * 16, 16)] = z
        return 0

    lax.fori_loop(0, nrows, body, 0)


@functools.partial(
    pl.kernel,
    out_type=jax.ShapeDtypeStruct((NW, NPAD), F32),
    mesh=_mesh,
    compiler_params=pltpu.CompilerParams(needs_layout_passes=False,
                                         use_tc_tiling_on_sc=False),
    scratch_types=[
        pltpu.VMEM((NPAD,), F32),
        pltpu.VMEM((EPW,), I32),
    ],
)
def k_deg(dst_hbm, out_hbm, cnt_v, dstA):
    c = lax.axis_index("c")
    s = lax.axis_index("s")
    wid = s * NC + c
    pltpu.sync_copy(dst_hbm.at[pl.ds(wid * EPW, EPW)], dstA)
    _zero_vec(cnt_v, NPAD)
    ones = jnp.ones((16,), F32)

    def chunk(g, _):
        for j in range(CH // 16):
            d16 = dstA[pl.ds(g * CH + j * 16, 16)]
            plsc.addupdate_scatter(cnt_v, [d16], ones)
        return 0

    lax.fori_loop(0, EPW // CH, chunk, 0)
    pltpu.sync_copy(cnt_v, out_hbm.at[wid])


def _make_agg(f, nch0, nch1):
    assert 16 * (nch0 + nch1) * CH == EPAD and nch0 % 2 == 0 and nch1 % 2 == 0

    @functools.partial(
        pl.kernel,
        out_type=jax.ShapeDtypeStruct((NC, NPAD, f), F32),
        mesh=_mesh,
        compiler_params=pltpu.CompilerParams(needs_layout_passes=False,
                                             use_tc_tiling_on_sc=False),
        scratch_types=[
            [pltpu.VMEM((CH,), I32)] * 2,
            [pltpu.VMEM((CH,), I32)] * 2,
            [pltpu.VMEM((CH, f), F32)] * 2,
            [pltpu.SemaphoreType.DMA] * 2,
            [pltpu.SemaphoreType.DMA] * 2,
            pltpu.VMEM_SHARED((NPAD, f), F32),
        ],
    )
    def k_agg(h_hbm, src_hbm, dst_hbm, out_hbm, srcb, dstb, rowsb, gsem,
              ssem, acc_sh):
        c = lax.axis_index("c")
        s = lax.axis_index("s")
        _zero_rows(rowsb[0], CH, f)

        def zs(i, _):
            pltpu.sync_copy(rowsb[0],
                            acc_sh.at[pl.ds(s * _NSLICE + i * CH, CH)])
            return 0

        lax.fori_loop(0, _NSLICE // CH, zs, 0)
        plsc.subcore_barrier()

        def run(nch, cb):
            NG = nch // 2

            def prep(b, g):
                base = (cb + g) * CH
                pltpu.sync_copy(src_hbm.at[pl.ds(base, CH)], srcb[b])
                pltpu.sync_copy(dst_hbm.at[pl.ds(base, CH)], dstb[b])
                pltpu.async_copy(h_hbm.at[srcb[b]], rowsb[b], gsem[b])

            for b in range(2):
                prep(b, b)

            def group(q, _):
                for b in range(2):
                    g = q * 2 + b
                    pltpu.make_async_copy(h_hbm.at[srcb[b]], rowsb[b],
                                          gsem[b]).wait()
                    pltpu.sync_copy(rowsb[b], acc_sh.at[dstb[b]], add=True)

                    @pl.when(q < NG - 1)
                    def _():
                        prep(b, g + 2)
                return 0

            lax.fori_loop(0, NG, group, 0)

        if nch0 == nch1:
            run(nch0, lax.axis_index("s") * NC * nch0
                + lax.axis_index("c") * nch0)
        else:
            @pl.when(c == 0)
            def _():
                run(nch0, s * nch0)

            @pl.when(c == 1)
            def _():
                run(nch1, NS * nch0 + s * nch1)

        plsc.subcore_barrier()
        pltpu.sync_copy(acc_sh.at[pl.ds(s * _NSLICE, _NSLICE)],
                        out_hbm.at[c, pl.ds(s * _NSLICE, _NSLICE)])

    return k_agg


k_agg128 = _make_agg(128, 140, 24)
k_agg16 = _make_agg(16, 106, 58)


@functools.partial(
    pl.kernel,
    out_type=(
        jax.ShapeDtypeStruct((4, NPAD, 64), F32),
        jax.ShapeDtypeStruct((4, NS, NPAD), F32),
    ),
    mesh=_mesh,
    compiler_params=pltpu.CompilerParams(needs_layout_passes=False,
                                         use_tc_tiling_on_sc=False),
    scratch_types=[
        pltpu.VMEM((NPAD,), F32),
        pltpu.VMEM((NPAD,), F32),
        pltpu.VMEM((NPAD,), F32),
        pltpu.VMEM((CH,), F32),
        [pltpu.VMEM((CH,), I32)] * 4,
        [pltpu.VMEM((CH,), I32)] * 4,
        [pltpu.VMEM((CH,), I32)] * 4,
        [pltpu.VMEM((CH, 64), F32)] * 4,
        [pltpu.SemaphoreType.DMA] * 4,
        [pltpu.SemaphoreType.DMA] * 4,
        pltpu.VMEM_SHARED((NPAD, 64), F32),
    ],
)
def k_gat(hflat_hbm, ascT_hbm, adcT_hbm, src_hbm, dst_hbm,
          numer_hbm, dparts_hbm,
          asc_v, adc_v, den_v, ea_v, srcb, dstb, idxb, rowsb, gsem, ssem,
          acc_sh):
    c = lax.axis_index("c")
    s = lax.axis_index("s")
    ebase = s * EPS
    NCHUNK = EPS // CH
    NG = NCHUNK // 4
    for hh in range(2):
        head = c * 2 + hh
        pltpu.sync_copy(ascT_hbm.at[head], asc_v)
        pltpu.sync_copy(adcT_hbm.at[head], adc_v)
        _zero_vec(den_v, NPAD)
        _zero_rows(rowsb[0], CH, 64)

        def zs(i, _):
            pltpu.sync_copy(rowsb[0],
                            acc_sh.at[pl.ds(s * _NSLICE + i * CH, CH)])
            return 0

        lax.fori_loop(0, _NSLICE // CH, zs, 0)
        plsc.subcore_barrier()

        def prep(b, g):
            base = ebase + g * CH
            pltpu.sync_copy(src_hbm.at[pl.ds(base, CH)], srcb[b])
            pltpu.sync_copy(dst_hbm.at[pl.ds(base, CH)], dstb[b])
            for j in range(CH // 16):
                sl = pl.ds(j * 16, 16)
                idxb[b][sl] = srcb[b][sl] * 4 + head
            pltpu.async_copy(hflat_hbm.at[idxb[b]], rowsb[b], gsem[b])

        for b in range(4):
            prep(b, b)

        def group(q, _):
            for b in range(4):
                g = q * 4 + b
                for j in range(CH // 16):
                    sl = pl.ds(j * 16, 16)
                    s16 = srcb[b][sl]
                    d16 = dstb[b][sl]
                    a = (plsc.load_gather(asc_v, [s16])
                         + plsc.load_gather(adc_v, [d16]))
                    a = jnp.where(a > 0, a, 0.2 * a)
                    eid = ebase + g * CH + j * 16 + lax.iota(I32, 16)
                    ea = jnp.where(eid < ETOT, jnp.exp(a), 0.0)
                    plsc.addupdate_scatter(den_v, [d16], ea)
                    ea_v[sl] = ea
                pltpu.make_async_copy(hflat_hbm.at[idxb[b]], rowsb[b],
                                      gsem[b]).wait()

                def rr(j, _):
                    ea16 = ea_v[pl.ds(j * 16, 16)]
                    for u in range(16):
                        r = j * 16 + u
                        e = ea16[u]
                        for k in range(4):
                            rowsb[b][r, pl.ds(k * 16, 16)] = (
                                rowsb[b][r, pl.ds(k * 16, 16)] * e)
                    return 0

                lax.fori_loop(0, CH // 16, rr, 0)
                pltpu.async_copy(rowsb[b], acc_sh.at[dstb[b]], ssem[b],
                                 add=True)
                pb = (b + 3) % 4

                def _wait_prep():
                    pltpu.make_async_copy(rowsb[pb], acc_sh.at[dstb[pb]],
                                          ssem[pb]).wait()
                    prep(pb, g + 3)

                if b == 0:
                    @pl.when(q >= 1)
                    def _():
                        _wait_prep()
                else:
                    @pl.when(q < NG - 1)
                    def _():
                        _wait_prep()
            return 0

        lax.fori_loop(0, NG, group, 0)
        for b in range(4):
            pltpu.make_async_copy(rowsb[b], acc_sh.at[dstb[b]],
                                  ssem[b]).wait()
        pltpu.sync_copy(den_v, dparts_hbm.at[head, s])
        plsc.subcore_barrier()
        pltpu.sync_copy(acc_sh.at[pl.ds(s * _NSLICE, _NSLICE)],
                        numer_hbm.at[head, pl.ds(s * _NSLICE, _NSLICE)])
        plsc.subcore_barrier()


def _row_ids(i):
    return i * BLK + lax.broadcasted_iota(I32, (BLK, 1), 0)


def _acc_stats(st_ref, i, y):
    @pl.when(i == 0)
    def _():
        st_ref[...] = jnp.zeros_like(st_ref)

    ym = jnp.where(_row_ids(i) < N, y, 0.0)
    s0 = jnp.sum(ym, axis=0)[None, :]
    s1 = jnp.sum(ym * ym, axis=0)[None, :]
    pad = jnp.zeros((6, y.shape[1]), F32)
    st_ref[...] = st_ref[...] + jnp.concatenate([s0, s1, pad], axis=0)


def _bn_apply(y, st_ref, g_ref, be_ref):
    m = st_ref[0:1, :] / N
    v = st_ref[1:2, :] / N - m * m
    yn = (y - m) * lax.rsqrt(v + 1e-5) * g_ref[...] + be_ref[...]
    return jnp.where(yn > 0, yn, 0.01 * yn)


def _full(shape):
    nd = len(shape)
    return pl.BlockSpec(shape, lambda i: (0,) * nd)


def t0_dinv(degp, x):
    def body(degp_ref, x_ref, dinv_ref, xs_ref):
        deg = jnp.sum(degp_ref[...], axis=0)
        di = jnp.where(deg > 0, lax.rsqrt(deg), 0.0)[:, None]
        dinv_ref[...] = di
        rows = lax.broadcasted_iota(I32, (NPAD, 1), 0)
        xs_ref[...] = jnp.where(rows < N, x_ref[...] * di, 0.0)

    return pl.pallas_call(
        body,
        out_shape=(jax.ShapeDtypeStruct((NPAD, 1), F32),
                   jax.ShapeDtypeStruct((NPAD, 128), F32)),
    )(degp, x)


def t1_y1(axp, dinv, W1):
    def body(axp_ref, dinv_ref, W1_ref, y1_ref, st_ref):
        i = pl.program_id(0)
        a = (axp_ref[0] + axp_ref[1]) * dinv_ref[...]
        y = jnp.dot(a, W1_ref[...], preferred_element_type=F32)
        y1_ref[...] = y
        _acc_stats(st_ref, i, y)

    return pl.pallas_call(
        body,
        grid=(NB,),
        in_specs=[
            pl.BlockSpec((NC, BLK, 128), lambda i: (0, i, 0)),
            pl.BlockSpec((BLK, 1), lambda i: (i, 0)),
            _full((128, 256)),
        ],
        out_specs=[
            pl.BlockSpec((BLK, 256), lambda i: (i, 0)),
            _full((8, 256)),
        ],
        out_shape=(jax.ShapeDtypeStruct((NPAD, 256), F32),
                   jax.ShapeDtypeStruct((8, 256), F32)),
    )(axp, dinv, W1)


def t2_h(y1, st1, x, Wr1, br1, g1, be1, W2, WaS, WaD):
    def body(y1_ref, st_ref, x_ref, Wr1_ref, br1_ref, g1_ref, be1_ref,
             W2_ref, WaS_ref, WaD_ref, h_ref, hh_ref, asc_ref, adc_ref):
        yn = _bn_apply(y1_ref[...], st_ref, g1_ref, be1_ref)
        h = yn + jnp.dot(x_ref[...], Wr1_ref[...],
                         preferred_element_type=F32) + br1_ref[...]
        h_ref[...] = h
        hw = jnp.dot(h, W2_ref[...], preferred_element_type=F32)
        hh_ref[...] = hw
        asc_ref[...] = jnp.dot(hw, WaS_ref[...], preferred_element_type=F32)
        adc_ref[...] = jnp.dot(hw, WaD_ref[...], preferred_element_type=F32)

    return pl.pallas_call(
        body,
        grid=(NB,),
        in_specs=[
            pl.BlockSpec((BLK, 256), lambda i: (i, 0)),
            _full((8, 256)),
            pl.BlockSpec((BLK, 128), lambda i: (i, 0)),
            _full((128, 256)),
            _full((1, 256)),
            _full((1, 256)),
            _full((1, 256)),
            _full((256, 256)),
            _full((256, 4)),
            _full((256, 4)),
        ],
        out_specs=[
            pl.BlockSpec((BLK, 256), lambda i: (i, 0)),
            pl.BlockSpec((BLK, 256), lambda i: (i, 0)),
            pl.BlockSpec((BLK, 4), lambda i: (i, 0)),
            pl.BlockSpec((BLK, 4), lambda i: (i, 0)),
        ],
        out_shape=(jax.ShapeDtypeStruct((NPAD, 256), F32),
                   jax.ShapeDtypeStruct((NPAD, 256), F32),
                   jax.ShapeDtypeStruct((NPAD, 4), F32),
                   jax.ShapeDtypeStruct((NPAD, 4), F32)),
    )(y1, st1, x, Wr1, br1, g1, be1, W2, WaS, WaD)


def t3_gat(numer, dparts):
    def body(nu_ref, dp_ref, y2_ref, st_ref):
        i = pl.program_id(0)
        den = jnp.sum(dp_ref[...], axis=1)
        rden = 1.0 / jnp.maximum(den, 1e-30)
        gat = jnp.sum(nu_ref[...] * rden[:, :, None], axis=0) * 0.25
        y2_ref[...] = gat
        _acc_stats(st_ref, i, gat)

    return pl.pallas_call(
        body,
        grid=(NB,),
        in_specs=[
            pl.BlockSpec((4, BLK, 64), lambda i: (0, i, 0)),
            pl.BlockSpec((4, NS, BLK), lambda i: (0, 0, i)),
        ],
        out_specs=[
            pl.BlockSpec((BLK, 64), lambda i: (i, 0)),
            _full((8, 64)),
        ],
        out_shape=(jax.ShapeDtypeStruct((NPAD, 64), F32),
                   jax.ShapeDtypeStruct((8, 64), F32)),
    )(numer, dparts)


def t4_l3in(y2, st2, h, Wr2, br2, g2, be2, W3, dinv):
    def body(y2_ref, st_ref, h_ref, Wr2_ref, br2_ref, g2_ref, be2_ref,
             W3_ref, dinv_ref, hB_ref, h3s_ref):
        i = pl.program_id(0)
        yn = _bn_apply(y2_ref[...], st_ref, g2_ref, be2_ref)
        hB = yn + jnp.dot(h_ref[...], Wr2_ref[...],
                          preferred_element_type=F32) + br2_ref[...]
        hB_ref[...] = hB
        h3 = jnp.dot(hB, W3_ref[...], preferred_element_type=F32)
        h3s_ref[...] = jnp.where(_row_ids(i) < N, h3 * dinv_ref[...], 0.0)

    return pl.pallas_call(
        body,
        grid=(NB,),
        in_specs=[
            pl.BlockSpec((BLK, 64), lambda i: (i, 0)),
            _full((8, 64)),
            pl.BlockSpec((BLK, 256), lambda i: (i, 0)),
            _full((256, 64)),
            _full((1, 64)),
            _full((1, 64)),
            _full((1, 64)),
            _full((64, 16)),
            pl.BlockSpec((BLK, 1), lambda i: (i, 0)),
        ],
        out_specs=[
            pl.BlockSpec((BLK, 64), lambda i: (i, 0)),
            pl.BlockSpec((BLK, 16), lambda i: (i, 0)),
        ],
        out_shape=(jax.ShapeDtypeStruct((NPAD, 64), F32),
                   jax.ShapeDtypeStruct((NPAD, 16), F32)),
    )(y2, st2, h, Wr2, br2, g2, be2, W3, dinv)


def t5_y3(p3, dinv):
    def body(p_ref, dinv_ref, y3_ref, st_ref):
        i = pl.program_id(0)
        y = (p_ref[0] + p_ref[1]) * dinv_ref[...]
        y3_ref[...] = y
        _acc_stats(st_ref, i, y)

    return pl.pallas_call(
        body,
        grid=(NB,),
        in_specs=[
            pl.BlockSpec((NC, BLK, 16), lambda i: (0, i, 0)),
            pl.BlockSpec((BLK, 1), lambda i: (i, 0)),
        ],
        out_specs=[
            pl.BlockSpec((BLK, 16), lambda i: (i, 0)),
            _full((8, 16)),
        ],
        out_shape=(jax.ShapeDtypeStruct((NPAD, 16), F32),
                   jax.ShapeDtypeStruct((8, 16), F32)),
    )(p3, dinv)


def t6_l4in(y3, st3, hB, Wr3, br3, g3, be3, dinv):
    def body(y3_ref, st_ref, hB_ref, Wr3_ref, br3_ref, g3_ref, be3_ref,
             dinv_ref, h4s_ref):
        i = pl.program_id(0)
        yn = _bn_apply(y3_ref[...], st_ref, g3_ref, be3_ref)
        hC = yn + jnp.dot(hB_ref[...], Wr3_ref[...],
                          preferred_element_type=F32) + br3_ref[...]
        h4s_ref[...] = jnp.where(_row_ids(i) < N, hC * dinv_ref[...], 0.0)

    return pl.pallas_call(
        body,
        grid=(NB,),
        in_specs=[
            pl.BlockSpec((BLK, 16), lambda i: (i, 0)),
            _full((8, 16)),
            pl.BlockSpec((BLK, 64), lambda i: (i, 0)),
            _full((64, 16)),
            _full((1, 16)),
            _full((1, 16)),
            _full((1, 16)),
            pl.BlockSpec((BLK, 1), lambda i: (i, 0)),
        ],
        out_specs=pl.BlockSpec((BLK, 16), lambda i: (i, 0)),
        out_shape=jax.ShapeDtypeStruct((NPAD, 16), F32),
    )(y3, st3, hB, Wr3, br3, g3, be3, dinv)


def t7_out(p4, dinv, W4, Wp, b4, bp):
    def body(p_ref, dinv_ref, W4_ref, Wp_ref, b4_ref, bp_ref, out_ref):
        a4 = (p_ref[0] + p_ref[1]) * dinv_ref[...]
        WW = jnp.dot(W4_ref[...], Wp_ref[...], preferred_element_type=F32)
        bb = jnp.dot(b4_ref[...], Wp_ref[...],
                     preferred_element_type=F32) + bp_ref[...]
        out_ref[...] = jnp.dot(a4, WW, preferred_element_type=F32) + bb

    return pl.pallas_call(
        body,
        grid=(NB,),
        in_specs=[
            pl.BlockSpec((NC, BLK, 16), lambda i: (0, i, 0)),
            pl.BlockSpec((BLK, 1), lambda i: (i, 0)),
            _full((16, 64)),
            _full((64, 64)),
            _full((1, 64)),
            _full((1, 64)),
        ],
        out_specs=pl.BlockSpec((BLK, 64), lambda i: (i, 0)),
        out_shape=jax.ShapeDtypeStruct((NPAD, 64), F32),
    )(p4, dinv, W4, Wp, b4, bp)


def kernel(x, edge_index, W1, b1, g1, be1, W2, asr, adt, b2, g2, be2,
           W3, b3, g3, be3, W4, b4, Wr1, br1, Wr2, br2, Wr3, br3, Wp, bp):
    ei = edge_index
    loop = jnp.arange(N, dtype=ei.dtype)
    padv = jnp.full((EPAD - ETOT,), N, dtype=ei.dtype)
    srcp = jnp.concatenate([ei[0], loop, padv])
    dstp = jnp.concatenate([ei[1], loop, padv])
    xpad = jnp.pad(x, ((0, NPAD - N), (0, 0)))

    eye4 = jnp.eye(4, dtype=F32)
    WaS = (asr[:, :, None] * eye4[:, None, :]).reshape(256, 4)
    WaD = (adt[:, :, None] * eye4[:, None, :]).reshape(256, 4)

    degp = k_deg(dstp)
    dinv, xs = t0_dinv(degp, xpad)
    axp = k_agg128(xs, srcp, dstp)
    y1, st1 = t1_y1(axp, dinv, W1)
    h, hw2, asc, adc = t2_h(y1, st1, xpad, Wr1, br1[None, :], g1[None, :],
                            be1[None, :], W2, WaS, WaD)
    hflat = hw2.reshape(4 * NPAD, 64)
    ascT = asc.T
    adcT = adc.T
    numer, dparts = k_gat(hflat, ascT, adcT, srcp, dstp)
    y2, st2 = t3_gat(numer, dparts)
    hB, h3s = t4_l3in(y2, st2, h, Wr2, br2[None, :], g2[None, :],
                      be2[None, :], W3, dinv)
    p3 = k_agg16(h3s, srcp, dstp)
    y3, st3 = t5_y3(p3, dinv)
    h4s = t6_l4in(y3, st3, hB, Wr3, br3[None, :], g3[None, :],
                  be3[None, :], dinv)
    p4 = k_agg16(h4s, srcp, dstp)
    out = t7_out(p4, dinv, W4, Wp, b4[None, :], bp[None, :])
    return out[:N]


# speedup vs baseline: 1.0343x; 1.0343x over previous
import functools

import jax
import jax.numpy as jnp
from jax import lax
from jax.experimental import pallas as pl
from jax.experimental.pallas import tpu as pltpu
from jax.experimental.pallas import tpu_sc as plsc

N = 10000
NPAD = 10240
E = 320000
ETOT = E + N
EPAD = 335872
NC, NS = 2, 16
NW = NC * NS
EPW = EPAD // NW
EPS = EPAD // NS
CH = 128
BLK = 1024
NB = NPAD // BLK
F32 = jnp.float32
I32 = jnp.int32

_mesh = plsc.VectorSubcoreMesh(core_axis_name="c", subcore_axis_name="s")

_NSLICE = NPAD // NS


def _zero_vec(ref, nwords):
    z = jnp.zeros((16,), F32)

    def body(i, _):
        ref[pl.ds(i * 16, 16)] = z
        return 0

    lax.fori_loop(0, nwords // 16, body, 0)


def _zero_rows(ref, nrows, f):
    z = jnp.zeros((16,), F32)

    def body(i, _):
        for k in range(f // 16):
            ref[i, pl.ds(k * 16, 16)] = z
        return 0

    lax.fori_loop(0, nrows, body, 0)


@functools.partial(
    pl.kernel,
    out_type=jax.ShapeDtypeStruct((NW, NPAD), F32),
    mesh=_mesh,
    compiler_params=pltpu.CompilerParams(needs_layout_passes=False,
                                         use_tc_tiling_on_sc=False),
    scratch_types=[
        pltpu.VMEM((NPAD,), F32),
        pltpu.VMEM((EPW,), I32),
    ],
)
def k_deg(dst_hbm, out_hbm, cnt_v, dstA):
    c = lax.axis_index("c")
    s = lax.axis_index("s")
    wid = s * NC + c
    pltpu.sync_copy(dst_hbm.at[pl.ds(wid * EPW, EPW)], dstA)
    _zero_vec(cnt_v, NPAD)
    ones = jnp.ones((16,), F32)

    def chunk(g, _):
        for j in range(CH // 16):
            d16 = dstA[pl.ds(g * CH + j * 16, 16)]
            plsc.addupdate_scatter(cnt_v, [d16], ones)
        return 0

    lax.fori_loop(0, EPW // CH, chunk, 0)
    pltpu.sync_copy(cnt_v, out_hbm.at[wid])


def _make_agg(f, nch0, nch1):
    assert 16 * (nch0 + nch1) * CH == EPAD and nch0 % 2 == 0 and nch1 % 2 == 0

    @functools.partial(
        pl.kernel,
        out_type=jax.ShapeDtypeStruct((NC, NPAD, f), F32),
        mesh=_mesh,
        compiler_params=pltpu.CompilerParams(needs_layout_passes=False,
                                             use_tc_tiling_on_sc=False),
        scratch_types=[
            [pltpu.VMEM((CH,), I32)] * 2,
            [pltpu.VMEM((CH,), I32)] * 2,
            [pltpu.VMEM((CH, f), F32)] * 2,
            [pltpu.SemaphoreType.DMA] * 2,
            [pltpu.SemaphoreType.DMA] * 2,
            pltpu.VMEM_SHARED((NPAD, f), F32),
        ],
    )
    def k_agg(h_hbm, src_hbm, dst_hbm, out_hbm, srcb, dstb, rowsb, gsem,
              ssem, acc_sh):
        c = lax.axis_index("c")
        s = lax.axis_index("s")
        _zero_rows(rowsb[0], CH, f)

        def zs(i, _):
            pltpu.sync_copy(rowsb[0],
                            acc_sh.at[pl.ds(s * _NSLICE + i * CH, CH)])
            return 0

        lax.fori_loop(0, _NSLICE // CH, zs, 0)
        plsc.subcore_barrier()

        def run(nch, cb):
            NG = nch // 2

            def prep(b, g):
                base = (cb + g) * CH
                pltpu.sync_copy(src_hbm.at[pl.ds(base, CH)], srcb[b])
                pltpu.sync_copy(dst_hbm.at[pl.ds(base, CH)], dstb[b])
                pltpu.async_copy(h_hbm.at[srcb[b]], rowsb[b], gsem[b])

            for b in range(2):
                prep(b, b)

            def group(q, _):
                for b in range(2):
                    g = q * 2 + b
                    pltpu.make_async_copy(h_hbm.at[srcb[b]], rowsb[b],
                                          gsem[b]).wait()
                    pltpu.sync_copy(rowsb[b], acc_sh.at[dstb[b]], add=True)

                    @pl.when(q < NG - 1)
                    def _():
                        prep(b, g + 2)
                return 0

            lax.fori_loop(0, NG, group, 0)

        if nch0 == nch1:
            run(nch0, lax.axis_index("s") * NC * nch0
                + lax.axis_index("c") * nch0)
        else:
            @pl.when(c == 0)
            def _():
                run(nch0, s * nch0)

            @pl.when(c == 1)
            def _():
                run(nch1, NS * nch0 + s * nch1)

        plsc.subcore_barrier()
        pltpu.sync_copy(acc_sh.at[pl.ds(s * _NSLICE, _NSLICE)],
                        out_hbm.at[c, pl.ds(s * _NSLICE, _NSLICE)])

    return k_agg


k_agg128 = _make_agg(128, 140, 24)
k_agg16 = _make_agg(16, 82, 82)


@functools.partial(
    pl.kernel,
    out_type=(
        jax.ShapeDtypeStruct((4, NPAD, 64), F32),
        jax.ShapeDtypeStruct((4, NS, NPAD), F32),
    ),
    mesh=_mesh,
    compiler_params=pltpu.CompilerParams(needs_layout_passes=False,
                                         use_tc_tiling_on_sc=False),
    scratch_types=[
        pltpu.VMEM((NPAD,), F32),
        pltpu.VMEM((NPAD,), F32),
        pltpu.VMEM((NPAD,), F32),
        pltpu.VMEM((CH,), F32),
        [pltpu.VMEM((CH,), I32)] * 4,
        [pltpu.VMEM((CH,), I32)] * 4,
        [pltpu.VMEM((CH,), I32)] * 4,
        [pltpu.VMEM((CH, 64), F32)] * 4,
        [pltpu.SemaphoreType.DMA] * 4,
        [pltpu.SemaphoreType.DMA] * 4,
        pltpu.VMEM_SHARED((NPAD, 64), F32),
    ],
)
def k_gat(hflat_hbm, ascT_hbm, adcT_hbm, src_hbm, dst_hbm,
          numer_hbm, dparts_hbm,
          asc_v, adc_v, den_v, ea_v, srcb, dstb, idxb, rowsb, gsem, ssem,
          acc_sh):
    c = lax.axis_index("c")
    s = lax.axis_index("s")
    ebase = s * EPS
    NCHUNK = EPS // CH
    NG = NCHUNK // 4
    for hh in range(2):
        head = c * 2 + hh
        pltpu.sync_copy(ascT_hbm.at[head], asc_v)
        pltpu.sync_copy(adcT_hbm.at[head], adc_v)
        _zero_vec(den_v, NPAD)
        _zero_rows(rowsb[0], CH, 64)

        def zs(i, _):
            pltpu.sync_copy(rowsb[0],
                            acc_sh.at[pl.ds(s * _NSLICE + i * CH, CH)])
            return 0

        lax.fori_loop(0, _NSLICE // CH, zs, 0)
        plsc.subcore_barrier()

        def prep(b, g):
            base = ebase + g * CH
            pltpu.sync_copy(src_hbm.at[pl.ds(base, CH)], srcb[b])
            pltpu.sync_copy(dst_hbm.at[pl.ds(base, CH)], dstb[b])
            for j in range(CH // 16):
                sl = pl.ds(j * 16, 16)
                idxb[b][sl] = srcb[b][sl] * 4 + head
            pltpu.async_copy(hflat_hbm.at[idxb[b]], rowsb[b], gsem[b])

        for b in range(4):
            prep(b, b)

        def group(q, _):
            for b in range(4):
                g = q * 4 + b
                for j in range(CH // 16):
                    sl = pl.ds(j * 16, 16)
                    s16 = srcb[b][sl]
                    d16 = dstb[b][sl]
                    a = (plsc.load_gather(asc_v, [s16])
                         + plsc.load_gather(adc_v, [d16]))
                    a = jnp.where(a > 0, a, 0.2 * a)
                    eid = ebase + g * CH + j * 16 + lax.iota(I32, 16)
                    ea = jnp.where(eid < ETOT, jnp.exp(a), 0.0)
                    plsc.addupdate_scatter(den_v, [d16], ea)
                    ea_v[sl] = ea
                pltpu.make_async_copy(hflat_hbm.at[idxb[b]], rowsb[b],
                                      gsem[b]).wait()

                def rr(j, _):
                    ea16 = ea_v[pl.ds(j * 16, 16)]
                    for u in range(16):
                        r = j * 16 + u
                        e = ea16[u]
                        for k in range(4):
                            rowsb[b][r, pl.ds(k * 16, 16)] = (
                                rowsb[b][r, pl.ds(k * 16, 16)] * e)
                    return 0

                lax.fori_loop(0, CH // 16, rr, 0)
                pltpu.async_copy(rowsb[b], acc_sh.at[dstb[b]], ssem[b],
                                 add=True)
                pb = (b + 3) % 4

                def _wait_prep():
                    pltpu.make_async_copy(rowsb[pb], acc_sh.at[dstb[pb]],
                                          ssem[pb]).wait()
                    prep(pb, g + 3)

                if b == 0:
                    @pl.when(q >= 1)
                    def _():
                        _wait_prep()
                else:
                    @pl.when(q < NG - 1)
                    def _():
                        _wait_prep()
            return 0

        lax.fori_loop(0, NG, group, 0)
        for b in range(4):
            pltpu.make_async_copy(rowsb[b], acc_sh.at[dstb[b]],
                                  ssem[b]).wait()
        pltpu.sync_copy(den_v, dparts_hbm.at[head, s])
        plsc.subcore_barrier()
        pltpu.sync_copy(acc_sh.at[pl.ds(s * _NSLICE, _NSLICE)],
                        numer_hbm.at[head, pl.ds(s * _NSLICE, _NSLICE)])
        plsc.subcore_barrier()


def _row_ids(i):
    return i * BLK + lax.broadcasted_iota(I32, (BLK, 1), 0)


def _acc_stats(st_ref, i, y):
    @pl.when(i == 0)
    def _():
        st_ref[...] = jnp.zeros_like(st_ref)

    ym = jnp.where(_row_ids(i) < N, y, 0.0)
    s0 = jnp.sum(ym, axis=0)[None, :]
    s1 = jnp.sum(ym * ym, axis=0)[None, :]
    pad = jnp.zeros((6, y.shape[1]), F32)
    st_ref[...] = st_ref[...] + jnp.concatenate([s0, s1, pad], axis=0)


def _bn_apply(y, st_ref, g_ref, be_ref):
    m = st_ref[0:1, :] / N
    v = st_ref[1:2, :] / N - m * m
    yn = (y - m) * lax.rsqrt(v + 1e-5) * g_ref[...] + be_ref[...]
    return jnp.where(yn > 0, yn, 0.01 * yn)


def _full(shape):
    nd = len(shape)
    return pl.BlockSpec(shape, lambda i: (0,) * nd)


def t0_dinv(degp, x):
    def body(degp_ref, x_ref, dinv_ref, xs_ref):
        deg = jnp.sum(degp_ref[...], axis=0)
        di = jnp.where(deg > 0, lax.rsqrt(deg), 0.0)[:, None]
        dinv_ref[...] = di
        rows = lax.broadcasted_iota(I32, (NPAD, 1), 0)
        xs_ref[...] = jnp.where(rows < N, x_ref[...] * di, 0.0)

    return pl.pallas_call(
        body,
        out_shape=(jax.ShapeDtypeStruct((NPAD, 1), F32),
                   jax.ShapeDtypeStruct((NPAD, 128), F32)),
    )(degp, x)


def t1_y1(axp, dinv, W1):
    def body(axp_ref, dinv_ref, W1_ref, y1_ref, st_ref):
        i = pl.program_id(0)
        a = (axp_ref[0] + axp_ref[1]) * dinv_ref[...]
        y = jnp.dot(a, W1_ref[...], preferred_element_type=F32)
        y1_ref[...] = y
        _acc_stats(st_ref, i, y)

    return pl.pallas_call(
        body,
        grid=(NB,),
        in_specs=[
            pl.BlockSpec((NC, BLK, 128), lambda i: (0, i, 0)),
            pl.BlockSpec((BLK, 1), lambda i: (i, 0)),
            _full((128, 256)),
        ],
        out_specs=[
            pl.BlockSpec((BLK, 256), lambda i: (i, 0)),
            _full((8, 256)),
        ],
        out_shape=(jax.ShapeDtypeStruct((NPAD, 256), F32),
                   jax.ShapeDtypeStruct((8, 256), F32)),
    )(axp, dinv, W1)


def t2_h(y1, st1, x, Wr1, br1, g1, be1, W2, WaS, WaD):
    def body(y1_ref, st_ref, x_ref, Wr1_ref, br1_ref, g1_ref, be1_ref,
             W2_ref, WaS_ref, WaD_ref, h_ref, hh_ref, asc_ref, adc_ref):
        yn = _bn_apply(y1_ref[...], st_ref, g1_ref, be1_ref)
        h = yn + jnp.dot(x_ref[...], Wr1_ref[...],
                         preferred_element_type=F32) + br1_ref[...]
        h_ref[...] = h
        hw = jnp.dot(h, W2_ref[...], preferred_element_type=F32)
        hh_ref[...] = hw
        asc_ref[...] = jnp.dot(hw, WaS_ref[...], preferred_element_type=F32)
        adc_ref[...] = jnp.dot(hw, WaD_ref[...], preferred_element_type=F32)

    return pl.pallas_call(
        body,
        grid=(NB,),
        in_specs=[
            pl.BlockSpec((BLK, 256), lambda i: (i, 0)),
            _full((8, 256)),
            pl.BlockSpec((BLK, 128), lambda i: (i, 0)),
            _full((128, 256)),
            _full((1, 256)),
            _full((1, 256)),
            _full((1, 256)),
            _full((256, 256)),
            _full((256, 4)),
            _full((256, 4)),
        ],
        out_specs=[
            pl.BlockSpec((BLK, 256), lambda i: (i, 0)),
            pl.BlockSpec((BLK, 256), lambda i: (i, 0)),
            pl.BlockSpec((BLK, 4), lambda i: (i, 0)),
            pl.BlockSpec((BLK, 4), lambda i: (i, 0)),
        ],
        out_shape=(jax.ShapeDtypeStruct((NPAD, 256), F32),
                   jax.ShapeDtypeStruct((NPAD, 256), F32),
                   jax.ShapeDtypeStruct((NPAD, 4), F32),
                   jax.ShapeDtypeStruct((NPAD, 4), F32)),
    )(y1, st1, x, Wr1, br1, g1, be1, W2, WaS, WaD)


def t3_gat(numer, dparts):
    def body(nu_ref, dp_ref, y2_ref, st_ref):
        i = pl.program_id(0)
        den = jnp.sum(dp_ref[...], axis=1)
        rden = 1.0 / jnp.maximum(den, 1e-30)
        gat = jnp.sum(nu_ref[...] * rden[:, :, None], axis=0) * 0.25
        y2_ref[...] = gat
        _acc_stats(st_ref, i, gat)

    return pl.pallas_call(
        body,
        grid=(NB,),
        in_specs=[
            pl.BlockSpec((4, BLK, 64), lambda i: (0, i, 0)),
            pl.BlockSpec((4, NS, BLK), lambda i: (0, 0, i)),
        ],
        out_specs=[
            pl.BlockSpec((BLK, 64), lambda i: (i, 0)),
            _full((8, 64)),
        ],
        out_shape=(jax.ShapeDtypeStruct((NPAD, 64), F32),
                   jax.ShapeDtypeStruct((8, 64), F32)),
    )(numer, dparts)


def t4_l3in(y2, st2, h, Wr2, br2, g2, be2, W3, dinv):
    def body(y2_ref, st_ref, h_ref, Wr2_ref, br2_ref, g2_ref, be2_ref,
             W3_ref, dinv_ref, hB_ref, h3s_ref):
        i = pl.program_id(0)
        yn = _bn_apply(y2_ref[...], st_ref, g2_ref, be2_ref)
        hB = yn + jnp.dot(h_ref[...], Wr2_ref[...],
                          preferred_element_type=F32) + br2_ref[...]
        hB_ref[...] = hB
        h3 = jnp.dot(hB, W3_ref[...], preferred_element_type=F32)
        h3s_ref[...] = jnp.where(_row_ids(i) < N, h3 * dinv_ref[...], 0.0)

    return pl.pallas_call(
        body,
        grid=(NB,),
        in_specs=[
            pl.BlockSpec((BLK, 64), lambda i: (i, 0)),
            _full((8, 64)),
            pl.BlockSpec((BLK, 256), lambda i: (i, 0)),
            _full((256, 64)),
            _full((1, 64)),
            _full((1, 64)),
            _full((1, 64)),
            _full((64, 16)),
            pl.BlockSpec((BLK, 1), lambda i: (i, 0)),
        ],
        out_specs=[
            pl.BlockSpec((BLK, 64), lambda i: (i, 0)),
            pl.BlockSpec((BLK, 16), lambda i: (i, 0)),
        ],
        out_shape=(jax.ShapeDtypeStruct((NPAD, 64), F32),
                   jax.ShapeDtypeStruct((NPAD, 16), F32)),
    )(y2, st2, h, Wr2, br2, g2, be2, W3, dinv)


def t5_y3(p3, dinv):
    def body(p_ref, dinv_ref, y3_ref, st_ref):
        i = pl.program_id(0)
        y = (p_ref[0] + p_ref[1]) * dinv_ref[...]
        y3_ref[...] = y
        _acc_stats(st_ref, i, y)

    return pl.pallas_call(
        body,
        grid=(NB,),
        in_specs=[
            pl.BlockSpec((NC, BLK, 16), lambda i: (0, i, 0)),
            pl.BlockSpec((BLK, 1), lambda i: (i, 0)),
        ],
        out_specs=[
            pl.BlockSpec((BLK, 16), lambda i: (i, 0)),
            _full((8, 16)),
        ],
        out_shape=(jax.ShapeDtypeStruct((NPAD, 16), F32),
                   jax.ShapeDtypeStruct((8, 16), F32)),
    )(p3, dinv)


def t6_l4in(y3, st3, hB, Wr3, br3, g3, be3, dinv):
    def body(y3_ref, st_ref, hB_ref, Wr3_ref, br3_ref, g3_ref, be3_ref,
             dinv_ref, h4s_ref):
        i = pl.program_id(0)
        yn = _bn_apply(y3_ref[...], st_ref, g3_ref, be3_ref)
        hC = yn + jnp.dot(hB_ref[...], Wr3_ref[...],
                          preferred_element_type=F32) + br3_ref[...]
        h4s_ref[...] = jnp.where(_row_ids(i) < N, hC * dinv_ref[...], 0.0)

    return pl.pallas_call(
        body,
        grid=(NB,),
        in_specs=[
            pl.BlockSpec((BLK, 16), lambda i: (i, 0)),
            _full((8, 16)),
            pl.BlockSpec((BLK, 64), lambda i: (i, 0)),
            _full((64, 16)),
            _full((1, 16)),
            _full((1, 16)),
            _full((1, 16)),
            pl.BlockSpec((BLK, 1), lambda i: (i, 0)),
        ],
        out_specs=pl.BlockSpec((BLK, 16), lambda i: (i, 0)),
        out_shape=jax.ShapeDtypeStruct((NPAD, 16), F32),
    )(y3, st3, hB, Wr3, br3, g3, be3, dinv)


def t7_out(p4, dinv, W4, Wp, b4, bp):
    def body(p_ref, dinv_ref, W4_ref, Wp_ref, b4_ref, bp_ref, out_ref):
        a4 = (p_ref[0] + p_ref[1]) * dinv_ref[...]
        WW = jnp.dot(W4_ref[...], Wp_ref[...], preferred_element_type=F32)
        bb = jnp.dot(b4_ref[...], Wp_ref[...],
                     preferred_element_type=F32) + bp_ref[...]
        out_ref[...] = jnp.dot(a4, WW, preferred_element_type=F32) + bb

    return pl.pallas_call(
        body,
        grid=(NB,),
        in_specs=[
            pl.BlockSpec((NC, BLK, 16), lambda i: (0, i, 0)),
            pl.BlockSpec((BLK, 1), lambda i: (i, 0)),
            _full((16, 64)),
            _full((64, 64)),
            _full((1, 64)),
            _full((1, 64)),
        ],
        out_specs=pl.BlockSpec((BLK, 64), lambda i: (i, 0)),
        out_shape=jax.ShapeDtypeStruct((NPAD, 64), F32),
    )(p4, dinv, W4, Wp, b4, bp)


def kernel(x, edge_index, W1, b1, g1, be1, W2, asr, adt, b2, g2, be2,
           W3, b3, g3, be3, W4, b4, Wr1, br1, Wr2, br2, Wr3, br3, Wp, bp):
    ei = edge_index
    loop = jnp.arange(N, dtype=ei.dtype)
    padv = jnp.full((EPAD - ETOT,), N, dtype=ei.dtype)
    srcp = jnp.concatenate([ei[0], loop, padv])
    dstp = jnp.concatenate([ei[1], loop, padv])
    xpad = jnp.pad(x, ((0, NPAD - N), (0, 0)))

    eye4 = jnp.eye(4, dtype=F32)
    WaS = (asr[:, :, None] * eye4[:, None, :]).reshape(256, 4)
    WaD = (adt[:, :, None] * eye4[:, None, :]).reshape(256, 4)

    degp = k_deg(dstp)
    dinv, xs = t0_dinv(degp, xpad)
    axp = k_agg128(xs, srcp, dstp)
    y1, st1 = t1_y1(axp, dinv, W1)
    h, hw2, asc, adc = t2_h(y1, st1, xpad, Wr1, br1[None, :], g1[None, :],
                            be1[None, :], W2, WaS, WaD)
    hflat = hw2.reshape(4 * NPAD, 64)
    ascT = asc.T
    adcT = adc.T
    numer, dparts = k_gat(hflat, ascT, adcT, srcp, dstp)
    y2, st2 = t3_gat(numer, dparts)
    hB, h3s = t4_l3in(y2, st2, h, Wr2, br2[None, :], g2[None, :],
                      be2[None, :], W3, dinv)
    p3 = k_agg16(h3s, srcp, dstp)
    y3, st3 = t5_y3(p3, dinv)
    h4s = t6_l4in(y3, st3, hB, Wr3, br3[None, :], g3[None, :],
                  be3[None, :], dinv)
    p4 = k_agg16(h4s, srcp, dstp)
    out = t7_out(p4, dinv, W4, Wp, b4[None, :], bp[None, :])
    return out[:N]
